# baseline (speedup 1.0000x reference)
import functools
import math

import jax
import jax.numpy as jnp
from jax import lax
from jax.experimental import pallas as pl
from jax.experimental.pallas import tpu as pltpu
from jax.experimental.pallas import tpu_sc as plsc

D_MODEL = 384
NHEAD = 8
DS = 4
OFFSET_SCALE = 4.0
B = 8
H = 56
W = 56
HQ = H // DS
WQ = W // DS
NQ = HQ * WQ
DH = D_MODEL // NHEAD
NPTS = B * NQ * NHEAD
NROWS = B * H * W

NW = 32
ROWS_PER_W = NPTS // NW
CHUNK = 56
NCHUNK = ROWS_PER_W // CHUNK


def _prep_body(fm_ref, wqT_ref, wkT_ref, wox_ref, woy_ref,
               qk_ref, idx_ref, w_ref):
    b = pl.program_id(0)
    fm = fm_ref[0]

    s_io = lax.broadcasted_iota(jnp.int32, (NQ, H * W), 1)
    q_io = lax.broadcasted_iota(jnp.int32, (NQ, H * W), 0)
    pgroup = (s_io // (W * DS)) * WQ + (s_io % W) // DS
    pool = jnp.where(pgroup == q_io, 1.0 / (DS * DS), 0.0).astype(jnp.float32)
    q_feat = jnp.dot(pool, fm, preferred_element_type=jnp.float32, precision=jax.lax.Precision.HIGHEST)

    queries = jnp.dot(q_feat, wqT_ref[...], preferred_element_type=jnp.float32, precision=jax.lax.Precision.HIGHEST)
    off_x = jnp.dot(q_feat, wox_ref[...],
                    preferred_element_type=jnp.float32) * OFFSET_SCALE
    off_y = jnp.dot(q_feat, woy_ref[...],
                    preferred_element_type=jnp.float32) * OFFSET_SCALE

    qq = lax.broadcasted_iota(jnp.int32, (NQ, NHEAD), 0)
    ref_x = (qq % WQ).astype(jnp.float32) * (2.0 / (WQ - 1)) - 1.0
    ref_y = (qq // WQ).astype(jnp.float32) * (2.0 / (HQ - 1)) - 1.0

    x = (ref_x + off_x + 1.0) * (0.5 * (W - 1))
    y = (ref_y + off_y + 1.0) * (0.5 * (H - 1))
    x0 = jnp.floor(x)
    y0 = jnp.floor(y)
    wx1 = x - x0
    wx0 = 1.0 - wx1
    wy1 = y - y0
    wy0 = 1.0 - wy1
    x0c = jnp.clip(x0, 0, W - 1).astype(jnp.int32)
    x1c = jnp.clip(x0 + 1.0, 0, W - 1).astype(jnp.int32)
    y0c = jnp.clip(y0, 0, H - 1).astype(jnp.int32)
    y1c = jnp.clip(y0 + 1.0, 0, H - 1).astype(jnp.int32)

    base = b * (H * W)
    i00 = base + y0c * W + x0c
    i01 = base + y0c * W + x1c
    i10 = base + y1c * W + x0c
    i11 = base + y1c * W + x1c
    idx_ref[0] = jnp.concatenate([i00, i01, i10, i11], axis=-1)
    w_ref[0] = jnp.concatenate([wy0 * wx0, wy0 * wx1, wy1 * wx0, wy1 * wx1],
                               axis=-1)

    scale = 1.0 / math.sqrt(float(DH))
    for h in range(NHEAD):
        qk_h = jnp.dot(queries[:, h * DH:(h + 1) * DH],
                       wkT_ref[h * DH:(h + 1) * DH, :],
                       preferred_element_type=jnp.float32, precision=jax.lax.Precision.HIGHEST)
        qk_ref[0, h] = qk_h * scale


def _prep_call(fm_hw3, wqT, wkT, wox, woy):
    return pl.pallas_call(
        _prep_body,
        grid=(B,),
        in_specs=[
            pl.BlockSpec((1, H * W, D_MODEL), lambda b: (b, 0, 0)),
            pl.BlockSpec((D_MODEL, D_MODEL), lambda b: (0, 0)),
            pl.BlockSpec((D_MODEL, D_MODEL), lambda b: (0, 0)),
            pl.BlockSpec((D_MODEL, NHEAD), lambda b: (0, 0)),
            pl.BlockSpec((D_MODEL, NHEAD), lambda b: (0, 0)),
        ],
        out_specs=[
            pl.BlockSpec((1, NHEAD, NQ, D_MODEL), lambda b: (b, 0, 0, 0)),
            pl.BlockSpec((1, NQ, 4 * NHEAD), lambda b: (b, 0, 0)),
            pl.BlockSpec((1, NQ, 4 * NHEAD), lambda b: (b, 0, 0)),
        ],
        out_shape=[
            jax.ShapeDtypeStruct((B, NHEAD, NQ, D_MODEL), jnp.float32),
            jax.ShapeDtypeStruct((B, NQ, 4 * NHEAD), jnp.int32),
            jax.ShapeDtypeStruct((B, NQ, 4 * NHEAD), jnp.float32),
        ],
    )(fm_hw3, wqT, wkT, wox, woy)


def _sc_gather_body(fm_hbm, idx_hbm, w_hbm, out_hbm, idx_v, w_v, bufs, out_v,
                    sem):
    cid = lax.axis_index("c")
    sid = lax.axis_index("s")
    wid = sid * 2 + cid
    base0 = wid * ROWS_PER_W

    @pl.loop(0, NCHUNK, unroll=1)
    def chunk_body(j):
        base = base0 + j * CHUNK
        for n in range(4):
            pltpu.sync_copy(idx_hbm.at[pl.ds(n * NPTS + base, CHUNK)],
                            idx_v.at[n])
            pltpu.sync_copy(w_hbm.at[pl.ds(n * NPTS + base, CHUNK)],
                            w_v.at[pl.ds(n * CHUNK, CHUNK)])
        copies = [
            pltpu.async_copy(fm_hbm.at[idx_v.at[n]], bufs.at[n], sem)
            for n in range(4)
        ]
        for cp in copies:
            cp.wait()

        @pl.loop(0, CHUNK, unroll=1)
        def row_body(r):
            wbc = [w_v[pl.ds(n * CHUNK + r, 16)][0] for n in range(4)]
            for cth in range(D_MODEL // 16):
                col = pl.ds(cth * 16, 16)
                acc = wbc[0] * bufs[0, r, col]
                acc += wbc[1] * bufs[1, r, col]
                acc += wbc[2] * bufs[2, r, col]
                acc += wbc[3] * bufs[3, r, col]
                out_v[r, col] = acc

        pltpu.sync_copy(out_v, out_hbm.at[pl.ds(base, CHUNK)])


def _sc_gather_call(fm_flat, idx4, w4):
    mesh = plsc.VectorSubcoreMesh(core_axis_name="c", subcore_axis_name="s")
    kern = functools.partial(
        pl.kernel,
        mesh=mesh,
        out_type=jax.ShapeDtypeStruct((NPTS, D_MODEL), jnp.float32),
        scratch_types=[
            pltpu.VMEM((4, CHUNK), jnp.int32),
            pltpu.VMEM((4 * CHUNK + 16,), jnp.float32),
            pltpu.VMEM((4, CHUNK, D_MODEL), jnp.float32),
            pltpu.VMEM((CHUNK, D_MODEL), jnp.float32),
            pltpu.SemaphoreType.DMA,
        ],
    )(_sc_gather_body)
    return kern(fm_flat, idx4, w4)


def _attn_body(smp_ref, qk_ref, wv_ref, out_ref):
    smp_p = [smp_ref[0, :, p, :] for p in range(NHEAD)]
    for h in range(NHEAD):
        qk_h = qk_ref[0, h]
        logit_cols = [
            jnp.sum(smp_p[p] * qk_h, axis=-1, keepdims=True)
            for p in range(NHEAD)
        ]
        logits = jnp.concatenate(logit_cols, axis=-1)
        m = jnp.max(logits, axis=-1, keepdims=True)
        e = jnp.exp(logits - m)
        attn = e / jnp.sum(e, axis=-1, keepdims=True)
        s_attn = attn[:, 0:1] * smp_p[0]
        for p in range(1, NHEAD):
            s_attn += attn[:, p:p + 1] * smp_p[p]
        out_h = jnp.dot(s_attn, wv_ref[:, h * DH:(h + 1) * DH],
                        preferred_element_type=jnp.float32, precision=jax.lax.Precision.HIGHEST)
        out_ref[0, :, h * DH:(h + 1) * DH] = out_h


def _attn_call(sampled4, qk, w_v):
    return pl.pallas_call(
        _attn_body,
        grid=(B,),
        in_specs=[
            pl.BlockSpec((1, NQ, NHEAD, D_MODEL), lambda b: (b, 0, 0, 0)),
            pl.BlockSpec((1, NHEAD, NQ, D_MODEL), lambda b: (b, 0, 0, 0)),
            pl.BlockSpec((D_MODEL, D_MODEL), lambda b: (0, 0)),
        ],
        out_specs=pl.BlockSpec((1, NQ, D_MODEL), lambda b: (b, 0, 0)),
        out_shape=jax.ShapeDtypeStruct((B, NQ, D_MODEL), jnp.float32),
    )(sampled4, qk, w_v)


def kernel(feature_map, W_q, W_k, W_v, W_off):
    fm_hw3 = feature_map.transpose(0, 2, 3, 1).reshape(B, H * W, D_MODEL)
    wqT = W_q.T
    wkT = W_k.T
    wox = W_off[:, 0::2]
    woy = W_off[:, 1::2]

    qk, idx_cat, w_cat = _prep_call(fm_hw3, wqT, wkT, wox, woy)

    idx4 = idx_cat.reshape(B, NQ, 4, NHEAD).transpose(2, 0, 1, 3).reshape(4 * NPTS)
    w4 = w_cat.reshape(B, NQ, 4, NHEAD).transpose(2, 0, 1, 3).reshape(4 * NPTS)

    fm_flat = fm_hw3.reshape(NROWS, D_MODEL)
    sampled = _sc_gather_call(fm_flat, idx4, w4)

    sampled4 = sampled.reshape(B, NQ, NHEAD, D_MODEL)
    return _attn_call(sampled4, qk, W_v)


# speedup vs baseline: 1.3148x; 1.3148x over previous
import functools
import math

import jax
import jax.numpy as jnp
from jax import lax
from jax.experimental import pallas as pl
from jax.experimental.pallas import tpu as pltpu
from jax.experimental.pallas import tpu_sc as plsc

D_MODEL = 384
NHEAD = 8
DS = 4
OFFSET_SCALE = 4.0
B = 8
H = 56
W = 56
HQ = H // DS
WQ = W // DS
NQ = HQ * WQ
DH = D_MODEL // NHEAD
NPTS = B * NQ * NHEAD
NROWS = B * H * W

NW = 32
ROWS_PER_W = NPTS // NW
CHUNK = 56
NCHUNK = ROWS_PER_W // CHUNK


def _prep_body(fm_ref, wqT_ref, wkbig_ref, wox_ref, woy_ref,
               qk_ref, idx_ref, w_ref):
    b = pl.program_id(0)
    fm = fm_ref[0]

    s_io = lax.broadcasted_iota(jnp.int32, (NQ, H * W), 1)
    q_io = lax.broadcasted_iota(jnp.int32, (NQ, H * W), 0)
    pgroup = (s_io // (W * DS)) * WQ + (s_io % W) // DS
    pool = jnp.where(pgroup == q_io, 1.0 / (DS * DS), 0.0).astype(jnp.float32)
    q_feat = jnp.dot(pool, fm, preferred_element_type=jnp.float32,
                     precision=jax.lax.Precision.HIGHEST)

    queries = jnp.dot(q_feat, wqT_ref[...],
                      preferred_element_type=jnp.float32)
    off_x = jnp.dot(q_feat, wox_ref[...],
                    preferred_element_type=jnp.float32) * OFFSET_SCALE
    off_y = jnp.dot(q_feat, woy_ref[...],
                    preferred_element_type=jnp.float32) * OFFSET_SCALE

    qq = lax.broadcasted_iota(jnp.int32, (NQ, NHEAD), 0)
    ref_x = (qq % WQ).astype(jnp.float32) * (2.0 / (WQ - 1)) - 1.0
    ref_y = (qq // WQ).astype(jnp.float32) * (2.0 / (HQ - 1)) - 1.0

    x = (ref_x + off_x + 1.0) * (0.5 * (W - 1))
    y = (ref_y + off_y + 1.0) * (0.5 * (H - 1))
    x0 = jnp.floor(x)
    y0 = jnp.floor(y)
    wx1 = x - x0
    wx0 = 1.0 - wx1
    wy1 = y - y0
    wy0 = 1.0 - wy1
    x0c = jnp.clip(x0, 0, W - 1).astype(jnp.int32)
    x1c = jnp.clip(x0 + 1.0, 0, W - 1).astype(jnp.int32)
    y0c = jnp.clip(y0, 0, H - 1).astype(jnp.int32)
    y1c = jnp.clip(y0 + 1.0, 0, H - 1).astype(jnp.int32)

    base = b * (H * W)
    i00 = base + y0c * W + x0c
    i01 = base + y0c * W + x1c
    i10 = base + y1c * W + x0c
    i11 = base + y1c * W + x1c
    idx_ref[0] = jnp.concatenate([i00, i01, i10, i11], axis=-1)
    w_ref[0] = jnp.concatenate([wy0 * wx0, wy0 * wx1, wy1 * wx0, wy1 * wx1],
                               axis=-1)

    scale = 1.0 / math.sqrt(float(DH))
    qk_ref[0] = jnp.dot(queries, wkbig_ref[...],
                        preferred_element_type=jnp.float32) * scale


def _prep_call(fm_hw3, wqT, wkbig, wox, woy):
    return pl.pallas_call(
        _prep_body,
        grid=(B,),
        in_specs=[
            pl.BlockSpec((1, H * W, D_MODEL), lambda b: (b, 0, 0)),
            pl.BlockSpec((D_MODEL, D_MODEL), lambda b: (0, 0)),
            pl.BlockSpec((D_MODEL, NHEAD * D_MODEL), lambda b: (0, 0)),
            pl.BlockSpec((D_MODEL, NHEAD), lambda b: (0, 0)),
            pl.BlockSpec((D_MODEL, NHEAD), lambda b: (0, 0)),
        ],
        out_specs=[
            pl.BlockSpec((1, NQ, NHEAD * D_MODEL), lambda b: (b, 0, 0)),
            pl.BlockSpec((1, NQ, 4 * NHEAD), lambda b: (b, 0, 0)),
            pl.BlockSpec((1, NQ, 4 * NHEAD), lambda b: (b, 0, 0)),
        ],
        out_shape=[
            jax.ShapeDtypeStruct((B, NQ, NHEAD * D_MODEL), jnp.float32),
            jax.ShapeDtypeStruct((B, NQ, 4 * NHEAD), jnp.int32),
            jax.ShapeDtypeStruct((B, NQ, 4 * NHEAD), jnp.float32),
        ],
    )(fm_hw3, wqT, wkbig, wox, woy)


def _sc_gather_body(fm_hbm, idx_hbm, w_hbm, out_hbm, idx_v, w_v, bufs, out_v,
                    sem):
    cid = lax.axis_index("c")
    sid = lax.axis_index("s")
    wid = sid * 2 + cid
    base0 = wid * ROWS_PER_W

    @pl.loop(0, NCHUNK, unroll=1)
    def chunk_body(j):
        base = base0 + j * CHUNK
        for n in range(4):
            pltpu.sync_copy(idx_hbm.at[pl.ds(n * NPTS + base, CHUNK)],
                            idx_v.at[n])
            pltpu.sync_copy(w_hbm.at[pl.ds(n * NPTS + base, CHUNK)],
                            w_v.at[pl.ds(n * CHUNK, CHUNK)])
        copies = [
            pltpu.async_copy(fm_hbm.at[idx_v.at[n]], bufs.at[n], sem)
            for n in range(4)
        ]
        for cp in copies:
            cp.wait()

        @pl.loop(0, CHUNK, unroll=1)
        def row_body(r):
            wbc = [w_v[pl.ds(n * CHUNK + r, 16)][0] for n in range(4)]
            for cth in range(D_MODEL // 16):
                col = pl.ds(cth * 16, 16)
                acc = wbc[0] * bufs[0, r, col]
                acc += wbc[1] * bufs[1, r, col]
                acc += wbc[2] * bufs[2, r, col]
                acc += wbc[3] * bufs[3, r, col]
                out_v[r, col] = acc

        pltpu.sync_copy(out_v, out_hbm.at[pl.ds(base, CHUNK)])


def _sc_gather_call(fm_flat, idx4, w4):
    mesh = plsc.VectorSubcoreMesh(core_axis_name="c", subcore_axis_name="s")
    kern = functools.partial(
        pl.kernel,
        mesh=mesh,
        out_type=jax.ShapeDtypeStruct((NPTS, D_MODEL), jnp.float32),
        scratch_types=[
            pltpu.VMEM((4, CHUNK), jnp.int32),
            pltpu.VMEM((4 * CHUNK + 16,), jnp.float32),
            pltpu.VMEM((4, CHUNK, D_MODEL), jnp.float32),
            pltpu.VMEM((CHUNK, D_MODEL), jnp.float32),
            pltpu.SemaphoreType.DMA,
        ],
    )(_sc_gather_body)
    return kern(fm_flat, idx4, w4)


def _attn_body(smp_ref, qk_ref, wvbd_ref, out_ref):
    smp_p = [smp_ref[0, p] for p in range(NHEAD)]
    s_cats = []
    for h in range(NHEAD):
        qk_h = qk_ref[0, :, h * D_MODEL:(h + 1) * D_MODEL]
        logit_cols = [
            jnp.sum(smp_p[p] * qk_h, axis=-1, keepdims=True)
            for p in range(NHEAD)
        ]
        logits = jnp.concatenate(logit_cols, axis=-1)
        m = jnp.max(logits, axis=-1, keepdims=True)
        e = jnp.exp(logits - m)
        attn = e / jnp.sum(e, axis=-1, keepdims=True)
        s_attn = attn[:, 0:1] * smp_p[0]
        for p in range(1, NHEAD):
            s_attn += attn[:, p:p + 1] * smp_p[p]
        s_cats.append(s_attn)
    s_cat = jnp.concatenate(s_cats, axis=-1)
    out_ref[0] = jnp.dot(s_cat, wvbd_ref[...],
                         preferred_element_type=jnp.float32)


def _attn_call(sampled4, qk, wvbd):
    return pl.pallas_call(
        _attn_body,
        grid=(B,),
        in_specs=[
            pl.BlockSpec((1, NHEAD, NQ, D_MODEL), lambda b: (b, 0, 0, 0)),
            pl.BlockSpec((1, NQ, NHEAD * D_MODEL), lambda b: (b, 0, 0)),
            pl.BlockSpec((NHEAD * D_MODEL, D_MODEL), lambda b: (0, 0)),
        ],
        out_specs=pl.BlockSpec((1, NQ, D_MODEL), lambda b: (b, 0, 0)),
        out_shape=jax.ShapeDtypeStruct((B, NQ, D_MODEL), jnp.float32),
    )(sampled4, qk, wvbd)


def kernel(feature_map, W_q, W_k, W_v, W_off):
    fm_hw3 = feature_map.transpose(0, 2, 3, 1).reshape(B, H * W, D_MODEL)
    wqT = W_q.T
    wox = W_off[:, 0::2]
    woy = W_off[:, 1::2]
    hmask_d = jnp.repeat(jnp.eye(NHEAD, dtype=jnp.float32), DH, axis=0)
    wkbig = (W_k.T[:, None, :] * hmask_d[:, :, None]).reshape(
        D_MODEL, NHEAD * D_MODEL)
    wvbd = (W_v[None, :, :] * hmask_d.T[:, None, :]).reshape(
        NHEAD * D_MODEL, D_MODEL)

    qk, idx_cat, w_cat = _prep_call(fm_hw3, wqT, wkbig, wox, woy)

    idx4 = idx_cat.reshape(B, NQ, 4, NHEAD).transpose(2, 0, 3, 1).reshape(4 * NPTS)
    w4 = w_cat.reshape(B, NQ, 4, NHEAD).transpose(2, 0, 3, 1).reshape(4 * NPTS)

    fm_flat = fm_hw3.reshape(NROWS, D_MODEL)
    sampled = _sc_gather_call(fm_flat, idx4, w4)

    sampled4 = sampled.reshape(B, NHEAD, NQ, D_MODEL)
    return _attn_call(sampled4, qk, wvbd)


# speedup vs baseline: 1.3384x; 1.0179x over previous
import functools
import math

import jax
import jax.numpy as jnp
from jax import lax
from jax.experimental import pallas as pl
from jax.experimental.pallas import tpu as pltpu
from jax.experimental.pallas import tpu_sc as plsc

D_MODEL = 384
NHEAD = 8
DS = 4
OFFSET_SCALE = 4.0
B = 8
H = 56
W = 56
HQ = H // DS
WQ = W // DS
NQ = HQ * WQ
DH = D_MODEL // NHEAD
NPTS = B * NQ * NHEAD
NROWS = B * H * W

NW = 32
ROWS_PER_W = NPTS // NW
CHUNK = 56
NCHUNK = ROWS_PER_W // CHUNK


def _prep_body(fm_ref, wqT_ref, wkbig_ref, wox_ref, woy_ref,
               qk_ref, idx_ref, w_ref):
    b = pl.program_id(0)
    fm = fm_ref[0]

    s_io = lax.broadcasted_iota(jnp.int32, (NQ, H * W), 1)
    q_io = lax.broadcasted_iota(jnp.int32, (NQ, H * W), 0)
    pgroup = (s_io // (W * DS)) * WQ + (s_io % W) // DS
    pool = jnp.where(pgroup == q_io, 1.0 / (DS * DS), 0.0).astype(jnp.float32)
    q_feat = jnp.dot(pool, fm, preferred_element_type=jnp.float32,
                     precision=jax.lax.Precision.HIGHEST)

    queries = jnp.dot(q_feat, wqT_ref[...],
                      preferred_element_type=jnp.float32)
    off_x = jnp.dot(q_feat, wox_ref[...],
                    preferred_element_type=jnp.float32) * OFFSET_SCALE
    off_y = jnp.dot(q_feat, woy_ref[...],
                    preferred_element_type=jnp.float32) * OFFSET_SCALE

    qq = lax.broadcasted_iota(jnp.int32, (NQ, NHEAD), 0)
    ref_x = (qq % WQ).astype(jnp.float32) * (2.0 / (WQ - 1)) - 1.0
    ref_y = (qq // WQ).astype(jnp.float32) * (2.0 / (HQ - 1)) - 1.0

    x = (ref_x + off_x + 1.0) * (0.5 * (W - 1))
    y = (ref_y + off_y + 1.0) * (0.5 * (H - 1))
    x0 = jnp.floor(x)
    y0 = jnp.floor(y)
    wx1 = x - x0
    wx0 = 1.0 - wx1
    wy1 = y - y0
    wy0 = 1.0 - wy1
    x0c = jnp.clip(x0, 0, W - 1).astype(jnp.int32)
    x1c = jnp.clip(x0 + 1.0, 0, W - 1).astype(jnp.int32)
    y0c = jnp.clip(y0, 0, H - 1).astype(jnp.int32)
    y1c = jnp.clip(y0 + 1.0, 0, H - 1).astype(jnp.int32)

    base = b * (H * W)
    i00 = base + y0c * W + x0c
    i01 = base + y0c * W + x1c
    i10 = base + y1c * W + x0c
    i11 = base + y1c * W + x1c
    idx_ref[0] = jnp.concatenate([i00, i01, i10, i11], axis=-1)
    w_ref[0] = jnp.concatenate([wy0 * wx0, wy0 * wx1, wy1 * wx0, wy1 * wx1],
                               axis=-1)

    scale = 1.0 / math.sqrt(float(DH))
    qk_ref[0] = jnp.dot(queries, wkbig_ref[...],
                        preferred_element_type=jnp.float32) * scale


def _prep_call(fm_hw3, wqT, wkbig, wox, woy):
    return pl.pallas_call(
        _prep_body,
        grid=(B,),
        in_specs=[
            pl.BlockSpec((1, H * W, D_MODEL), lambda b: (b, 0, 0)),
            pl.BlockSpec((D_MODEL, D_MODEL), lambda b: (0, 0)),
            pl.BlockSpec((D_MODEL, NHEAD * D_MODEL), lambda b: (0, 0)),
            pl.BlockSpec((D_MODEL, NHEAD), lambda b: (0, 0)),
            pl.BlockSpec((D_MODEL, NHEAD), lambda b: (0, 0)),
        ],
        out_specs=[
            pl.BlockSpec((1, NQ, NHEAD * D_MODEL), lambda b: (b, 0, 0)),
            pl.BlockSpec((1, NQ, 4 * NHEAD), lambda b: (b, 0, 0)),
            pl.BlockSpec((1, NQ, 4 * NHEAD), lambda b: (b, 0, 0)),
        ],
        out_shape=[
            jax.ShapeDtypeStruct((B, NQ, NHEAD * D_MODEL), jnp.float32),
            jax.ShapeDtypeStruct((B, NQ, 4 * NHEAD), jnp.int32),
            jax.ShapeDtypeStruct((B, NQ, 4 * NHEAD), jnp.float32),
        ],
    )(fm_hw3, wqT, wkbig, wox, woy)


def _sc_gather_body(fm_hbm, idx_hbm, w_hbm, out_hbm, idx_a, idx_b, w_all,
                    buf_a, buf_b, out_v, sem_a, sem_b):
    cid = lax.axis_index("c")
    sid = lax.axis_index("s")
    wid = sid * 2 + cid
    base0 = wid * ROWS_PER_W

    for n in range(4):
        pltpu.sync_copy(w_hbm.at[pl.ds(n * NPTS + base0, ROWS_PER_W)],
                        w_all.at[pl.ds(n * ROWS_PER_W, ROWS_PER_W)])

    def issue(j, pair, idxbuf, buf, sem):
        for k in range(2):
            n = pair * 2 + k
            pltpu.sync_copy(
                idx_hbm.at[pl.ds(n * NPTS + base0 + j * CHUNK, CHUNK)],
                idxbuf.at[k])
            pltpu.async_copy(fm_hbm.at[idxbuf.at[k]], buf.at[k], sem)

    issue(0, 0, idx_a, buf_a, sem_a)

    @pl.loop(0, NCHUNK, unroll=1)
    def chunk_body(j):
        jbase = j * CHUNK
        issue(j, 1, idx_b, buf_b, sem_b)
        for k in range(2):
            pltpu.make_async_copy(fm_hbm.at[pl.ds(0, CHUNK)], buf_a.at[k],
                                  sem_a).wait()

        @pl.loop(0, CHUNK, unroll=1)
        def row0_body(r):
            w0 = w_all[pl.ds(jbase + r, 16)][0]
            w1 = w_all[pl.ds(ROWS_PER_W + jbase + r, 16)][0]
            for cth in range(D_MODEL // 16):
                col = pl.ds(cth * 16, 16)
                out_v[r, col] = w0 * buf_a[0, r, col] + w1 * buf_a[1, r, col]

        @pl.when(j + 1 < NCHUNK)
        def _():
            issue(j + 1, 0, idx_a, buf_a, sem_a)

        for k in range(2):
            pltpu.make_async_copy(fm_hbm.at[pl.ds(0, CHUNK)], buf_b.at[k],
                                  sem_b).wait()

        @pl.loop(0, CHUNK, unroll=1)
        def row1_body(r):
            w2 = w_all[pl.ds(2 * ROWS_PER_W + jbase + r, 16)][0]
            w3 = w_all[pl.ds(3 * ROWS_PER_W + jbase + r, 16)][0]
            for cth in range(D_MODEL // 16):
                col = pl.ds(cth * 16, 16)
                acc = w2 * buf_b[0, r, col] + w3 * buf_b[1, r, col]
                out_v[r, col] += acc

        pltpu.sync_copy(out_v, out_hbm.at[pl.ds(base0 + jbase, CHUNK)])


def _sc_gather_call(fm_flat, idx4, w4):
    mesh = plsc.VectorSubcoreMesh(core_axis_name="c", subcore_axis_name="s")
    kern = functools.partial(
        pl.kernel,
        mesh=mesh,
        out_type=jax.ShapeDtypeStruct((NPTS, D_MODEL), jnp.float32),
        scratch_types=[
            pltpu.VMEM((2, CHUNK), jnp.int32),
            pltpu.VMEM((2, CHUNK), jnp.int32),
            pltpu.VMEM((4 * ROWS_PER_W + 16,), jnp.float32),
            pltpu.VMEM((2, CHUNK, D_MODEL), jnp.float32),
            pltpu.VMEM((2, CHUNK, D_MODEL), jnp.float32),
            pltpu.VMEM((CHUNK, D_MODEL), jnp.float32),
            pltpu.SemaphoreType.DMA,
            pltpu.SemaphoreType.DMA,
        ],
    )(_sc_gather_body)
    return kern(fm_flat, idx4, w4)


def _attn_body(smp_ref, qk_ref, wvbd_ref, out_ref):
    smp_p = [smp_ref[0, p] for p in range(NHEAD)]
    s_cats = []
    for h in range(NHEAD):
        qk_h = qk_ref[0, :, h * D_MODEL:(h + 1) * D_MODEL]
        logit_cols = [
            jnp.sum(smp_p[p] * qk_h, axis=-1, keepdims=True)
            for p in range(NHEAD)
        ]
        logits = jnp.concatenate(logit_cols, axis=-1)
        m = jnp.max(logits, axis=-1, keepdims=True)
        e = jnp.exp(logits - m)
        attn = e / jnp.sum(e, axis=-1, keepdims=True)
        s_attn = attn[:, 0:1] * smp_p[0]
        for p in range(1, NHEAD):
            s_attn += attn[:, p:p + 1] * smp_p[p]
        s_cats.append(s_attn)
    s_cat = jnp.concatenate(s_cats, axis=-1)
    out_ref[0] = jnp.dot(s_cat, wvbd_ref[...],
                         preferred_element_type=jnp.float32)


def _attn_call(sampled4, qk, wvbd):
    return pl.pallas_call(
        _attn_body,
        grid=(B,),
        in_specs=[
            pl.BlockSpec((1, NHEAD, NQ, D_MODEL), lambda b: (b, 0, 0, 0)),
            pl.BlockSpec((1, NQ, NHEAD * D_MODEL), lambda b: (b, 0, 0)),
            pl.BlockSpec((NHEAD * D_MODEL, D_MODEL), lambda b: (0, 0)),
        ],
        out_specs=pl.BlockSpec((1, NQ, D_MODEL), lambda b: (b, 0, 0)),
        out_shape=jax.ShapeDtypeStruct((B, NQ, D_MODEL), jnp.float32),
    )(sampled4, qk, wvbd)


def kernel(feature_map, W_q, W_k, W_v, W_off):
    fm_hw3 = feature_map.transpose(0, 2, 3, 1).reshape(B, H * W, D_MODEL)
    wqT = W_q.T
    wox = W_off[:, 0::2]
    woy = W_off[:, 1::2]
    hmask_d = jnp.repeat(jnp.eye(NHEAD, dtype=jnp.float32), DH, axis=0)
    wkbig = (W_k.T[:, None, :] * hmask_d[:, :, None]).reshape(
        D_MODEL, NHEAD * D_MODEL)
    wvbd = (W_v[None, :, :] * hmask_d.T[:, None, :]).reshape(
        NHEAD * D_MODEL, D_MODEL)

    qk, idx_cat, w_cat = _prep_call(fm_hw3, wqT, wkbig, wox, woy)

    idx4 = idx_cat.reshape(B, NQ, 4, NHEAD).transpose(2, 0, 3, 1).reshape(4 * NPTS)
    w4 = w_cat.reshape(B, NQ, 4, NHEAD).transpose(2, 0, 3, 1).reshape(4 * NPTS)

    fm_flat = fm_hw3.reshape(NROWS, D_MODEL)
    sampled = _sc_gather_call(fm_flat, idx4, w4)

    sampled4 = sampled.reshape(B, NHEAD, NQ, D_MODEL)
    return _attn_call(sampled4, qk, wvbd)
